# baseline (speedup 1.0000x reference)
import jax
import jax.numpy as jnp
from jax import lax
from jax.experimental import pallas as pl
from jax.experimental.pallas import tpu as pltpu
from jax.experimental.pallas import tpu_sc as plsc

N = 100000
D_IN = 256
D_OUT = 256
G = 512
BN = 2000
GRID = N // BN
NC = 2
NS = 16
CH = 120
ROWS_PER_TILE = 6240
NCHUNK = ROWS_PER_TILE // CH
TAIL0 = NS * ROWS_PER_TILE
TAILCH = 80
CW = 128


def _stage_a_body(x_ref, idx_ref, w_ref, b_ref, wg_ref, bg_ref,
                  h_ref, gnum_ref, denom_ref):
    x = x_ref[...]
    h_ref[...] = jnp.tanh(
        lax.dot_general(x, w_ref[...], (((1,), (1,)), ((), ())),
                        preferred_element_type=jnp.float32) + b_ref[...])
    gfull = lax.dot_general(x, wg_ref[...], (((1,), (1,)), ((), ())),
                            preferred_element_type=jnp.float32)
    gnum = jnp.exp(gfull[:, :1] + bg_ref[0, 0])
    gnum_ref[...] = gnum
    seg = idx_ref[:, :1]
    oh = (lax.broadcasted_iota(jnp.int32, (BN, G), 1) == seg
          ).astype(jnp.float32)
    part = lax.dot_general(oh, gnum, (((0,), (0,)), ((), ())),
                           preferred_element_type=jnp.float32)

    @pl.when(pl.program_id(0) == 0)
    def _():
        denom_ref[...] = part

    @pl.when(pl.program_id(0) != 0)
    def _():
        denom_ref[...] += part


def _stage_a(x, index, w, b2, wg, bg2):
    return pl.pallas_call(
        _stage_a_body,
        grid=(GRID,),
        in_specs=[
            pl.BlockSpec((BN, D_IN), lambda i: (i, 0)),
            pl.BlockSpec((BN, 128), lambda i: (i, 0)),
            pl.BlockSpec((D_OUT, D_IN), lambda i: (0, 0)),
            pl.BlockSpec((1, D_OUT), lambda i: (0, 0)),
            pl.BlockSpec((128, D_IN), lambda i: (0, 0)),
            pl.BlockSpec(memory_space=pltpu.SMEM),
        ],
        out_specs=[
            pl.BlockSpec((BN, D_OUT), lambda i: (i, 0)),
            pl.BlockSpec((BN, 1), lambda i: (i, 0)),
            pl.BlockSpec((G, 1), lambda i: (0, 0)),
        ],
        out_shape=[
            jax.ShapeDtypeStruct((N, D_OUT), jnp.float32),
            jax.ShapeDtypeStruct((N, 1), jnp.float32),
            jax.ShapeDtypeStruct((G, 1), jnp.float32),
        ],
    )(x, index, w, b2, wg, bg2)


def _stage_b_body(h_ref, idx_ref, gnum_ref, denom_ref, src_ref):
    d = denom_ref[...]
    rec = jnp.where(d > 0.0, 1.0 / d, 0.0)
    seg = idx_ref[:, :1]
    oh = (lax.broadcasted_iota(jnp.int32, (BN, G), 1) == seg
          ).astype(jnp.float32)
    gathered = lax.dot_general(oh, rec, (((1,), (0,)), ((), ())),
                               preferred_element_type=jnp.float32)
    src_ref[...] = (gnum_ref[...] * gathered) * h_ref[...]


def _stage_b(h, index, gnum, denom):
    return pl.pallas_call(
        _stage_b_body,
        grid=(GRID,),
        in_specs=[
            pl.BlockSpec((BN, D_OUT), lambda i: (i, 0)),
            pl.BlockSpec((BN, 128), lambda i: (i, 0)),
            pl.BlockSpec((BN, 1), lambda i: (i, 0)),
            pl.BlockSpec((G, 1), lambda i: (0, 0)),
        ],
        out_specs=pl.BlockSpec((BN, D_OUT), lambda i: (i, 0)),
        out_shape=jax.ShapeDtypeStruct((N, D_OUT), jnp.float32),
    )(h, index, gnum, denom)


SLICE = G * CW // NS


def _sc_body(src_hbm, idx_hbm, out_hbm, sbuf, ibuf, fbuf, vbuf,
             ftail, zbuf, shared, sem):
    c = lax.axis_index("c")
    s = lax.axis_index("s")
    col0 = c * CW
    iotas = [lax.iota(jnp.int32, 16) + 16 * cg for cg in range(CW // 16)]
    zero16 = jnp.zeros((16,), jnp.float32)

    def _zero(i, carry):
        zbuf[pl.ds(i * 16, 16)] = zero16
        return carry

    lax.fori_loop(0, SLICE // 16, _zero, 0)
    pltpu.sync_copy(zbuf, shared.at[pl.ds(s * SLICE, SLICE)])
    plsc.subcore_barrier()

    def _copies(r0, slot):
        return (
            pltpu.make_async_copy(
                src_hbm.at[pl.ds(r0, CH), pl.ds(col0, CW)],
                sbuf.at[slot], sem.at[slot]),
            pltpu.make_async_copy(
                idx_hbm.at[pl.ds(r0, CH), pl.ds(col0, CW)],
                ibuf.at[slot], sem.at[slot]),
        )

    def _flatten(sb, ib, fb, nrows):
        def _rows(g, carry2):
            base = g * 8
            for u in range(8):
                row = base + u
                for cg in range(CW // 16):
                    off = row * CW + cg * 16
                    iv = ib[row, pl.ds(cg * 16, 16)]
                    fb[pl.ds(off, 16)] = (iv << 7) + iotas[cg]
                    vbuf[pl.ds(off, 16)] = sb[row, pl.ds(cg * 16, 16)]
            return carry2

        lax.fori_loop(0, nrows // 8, _rows, 0)

    for cp in _copies(s * ROWS_PER_TILE, 0):
        cp.start()

    def _chunk(k, carry):
        slot = lax.rem(k, 2)

        @pl.when(k + 1 < NCHUNK)
        def _():
            for cp in _copies(s * ROWS_PER_TILE + (k + 1) * CH,
                              lax.rem(k + 1, 2)):
                cp.start()

        for cp in _copies(s * ROWS_PER_TILE + k * CH, slot):
            cp.wait()
        _flatten(sbuf.at[slot], ibuf.at[slot], fbuf, CH)
        pltpu.sync_copy(vbuf, shared.at[fbuf], add=True)
        return carry

    lax.fori_loop(0, NCHUNK, _chunk, 0)

    @pl.when(s < 2)
    def _():
        t0 = TAIL0 + s * TAILCH
        pltpu.sync_copy(src_hbm.at[pl.ds(t0, TAILCH), pl.ds(col0, CW)],
                        sbuf.at[0, pl.ds(0, TAILCH)])
        pltpu.sync_copy(idx_hbm.at[pl.ds(t0, TAILCH), pl.ds(col0, CW)],
                        ibuf.at[0, pl.ds(0, TAILCH)])
        _flatten(sbuf.at[0], ibuf.at[0], ftail, TAILCH)
        pltpu.sync_copy(vbuf.at[pl.ds(0, TAILCH * CW)],
                        shared.at[ftail], add=True)

    plsc.subcore_barrier()
    pltpu.sync_copy(shared.at[pl.ds(s * SLICE, SLICE)],
                    out_hbm.at[c, pl.ds(s * SLICE, SLICE)])


def _stage_c(src, index):
    mesh = plsc.VectorSubcoreMesh(core_axis_name="c", subcore_axis_name="s")
    f = pl.kernel(
        _sc_body,
        out_type=jax.ShapeDtypeStruct((NC, G * CW), jnp.float32),
        mesh=mesh,
        compiler_params=pltpu.CompilerParams(needs_layout_passes=False),
        scratch_types=[
            pltpu.VMEM((2, CH, CW), jnp.float32),
            pltpu.VMEM((2, CH, CW), jnp.int32),
            pltpu.VMEM((CH * CW,), jnp.int32),
            pltpu.VMEM((CH * CW,), jnp.float32),
            pltpu.VMEM((TAILCH * CW,), jnp.int32),
            pltpu.VMEM((SLICE,), jnp.float32),
            pltpu.VMEM_SHARED((G * CW,), jnp.float32),
            pltpu.SemaphoreType.DMA((2,)),
        ],
    )
    return f(src, index)


def _stage_d_body(p_ref, o_ref):
    o_ref[...] = jnp.tanh(p_ref[0])


def _stage_d(partial3):
    return pl.pallas_call(
        _stage_d_body,
        grid=(NC,),
        in_specs=[pl.BlockSpec((1, G, CW), lambda c: (c, 0, 0))],
        out_specs=pl.BlockSpec((G, CW), lambda c: (0, c)),
        out_shape=jax.ShapeDtypeStruct((G, D_OUT), jnp.float32),
    )(partial3)


def kernel(x, n_graph, index, Wg, bg, W, b):
    b2 = b.reshape(1, D_OUT)
    bg2 = bg.reshape(1, 1)
    wgp = jnp.pad(Wg, ((0, 127), (0, 0)))
    h, gnum, denom = _stage_a(x, index, W, b2, wgp, bg2)
    src = _stage_b(h, index, gnum, denom)
    partial = _stage_c(src, index)
    return _stage_d(partial.reshape(NC, G, CW))


# speedup vs baseline: 1.5158x; 1.5158x over previous
import jax
import jax.numpy as jnp
from jax import lax
from jax.experimental import pallas as pl
from jax.experimental.pallas import tpu as pltpu
from jax.experimental.pallas import tpu_sc as plsc

N = 100000
D_IN = 256
D_OUT = 256
G = 512
BN = 2000
GRID = N // BN
NC = 2
NS = 16
CH = 120
ROWS_PER_TILE = 6240
NCHUNK = ROWS_PER_TILE // CH
TAIL0 = NS * ROWS_PER_TILE
TAILCH = 80
CW = 128


def _stage_a_body(x_ref, idx_ref, w_ref, b_ref, wg_ref, bg_ref,
                  h_ref, gnum_ref, denom_ref):
    x = x_ref[...]
    h_ref[...] = jnp.tanh(
        lax.dot_general(x, w_ref[...], (((1,), (1,)), ((), ())),
                        preferred_element_type=jnp.float32) + b_ref[...])
    gfull = lax.dot_general(x, wg_ref[...], (((1,), (1,)), ((), ())),
                            preferred_element_type=jnp.float32)
    gnum = jnp.exp(gfull[:, :1] + bg_ref[0, 0])
    gnum_ref[...] = gnum
    seg = idx_ref[:, :1]
    oh = (lax.broadcasted_iota(jnp.int32, (BN, G), 1) == seg
          ).astype(jnp.float32)
    part = lax.dot_general(oh, gnum, (((0,), (0,)), ((), ())),
                           preferred_element_type=jnp.float32)

    @pl.when(pl.program_id(0) == 0)
    def _():
        denom_ref[...] = part

    @pl.when(pl.program_id(0) != 0)
    def _():
        denom_ref[...] += part


def _stage_a(x, index, w, b2, wg, bg2):
    return pl.pallas_call(
        _stage_a_body,
        grid=(GRID,),
        in_specs=[
            pl.BlockSpec((BN, D_IN), lambda i: (i, 0)),
            pl.BlockSpec((BN, 128), lambda i: (i, 0)),
            pl.BlockSpec((D_OUT, D_IN), lambda i: (0, 0)),
            pl.BlockSpec((1, D_OUT), lambda i: (0, 0)),
            pl.BlockSpec((128, D_IN), lambda i: (0, 0)),
            pl.BlockSpec(memory_space=pltpu.SMEM),
        ],
        out_specs=[
            pl.BlockSpec((BN, D_OUT), lambda i: (i, 0)),
            pl.BlockSpec((BN, 1), lambda i: (i, 0)),
            pl.BlockSpec((G, 1), lambda i: (0, 0)),
        ],
        out_shape=[
            jax.ShapeDtypeStruct((N, D_OUT), jnp.float32),
            jax.ShapeDtypeStruct((N, 1), jnp.float32),
            jax.ShapeDtypeStruct((G, 1), jnp.float32),
        ],
    )(x, index, w, b2, wg, bg2)


def _stage_b_body(h_ref, idx_ref, gnum_ref, denom_ref, src_ref):
    d = denom_ref[...]
    rec = jnp.where(d > 0.0, 1.0 / d, 0.0)
    seg = idx_ref[:, :1]
    oh = (lax.broadcasted_iota(jnp.int32, (BN, G), 1) == seg
          ).astype(jnp.float32)
    gathered = lax.dot_general(oh, rec, (((1,), (0,)), ((), ())),
                               preferred_element_type=jnp.float32)
    src_ref[...] = (gnum_ref[...] * gathered) * h_ref[...]


def _stage_b(h, index, gnum, denom):
    return pl.pallas_call(
        _stage_b_body,
        grid=(GRID,),
        in_specs=[
            pl.BlockSpec((BN, D_OUT), lambda i: (i, 0)),
            pl.BlockSpec((BN, 128), lambda i: (i, 0)),
            pl.BlockSpec((BN, 1), lambda i: (i, 0)),
            pl.BlockSpec((G, 1), lambda i: (0, 0)),
        ],
        out_specs=pl.BlockSpec((BN, D_OUT), lambda i: (i, 0)),
        out_shape=jax.ShapeDtypeStruct((N, D_OUT), jnp.float32),
    )(h, index, gnum, denom)


def _sc_body(src_hbm, idx_hbm, out_hbm, sbuf, ibuf, acc, sem):
    c = lax.axis_index("c")
    s = lax.axis_index("s")
    col0 = c * CW
    iotas = [lax.iota(jnp.int32, 16) + 16 * cg for cg in range(CW // 16)]
    zero16 = jnp.zeros((16,), jnp.float32)

    def _zero(i, carry):
        for u in range(8):
            acc[pl.ds((i * 8 + u) * 16, 16)] = zero16
        return carry

    lax.fori_loop(0, G * CW // 128, _zero, 0)

    def _copies(r0, slot):
        return (
            pltpu.make_async_copy(
                src_hbm.at[pl.ds(r0, CH), pl.ds(col0, CW)],
                sbuf.at[slot], sem.at[slot]),
            pltpu.make_async_copy(
                idx_hbm.at[pl.ds(r0, CH), pl.ds(col0, CW)],
                ibuf.at[slot], sem.at[slot]),
        )

    def _compute(slot, nrows):
        def _rows(g, carry2):
            base = g * 8
            for u in range(8):
                row = base + u
                for cg in range(CW // 16):
                    val = sbuf[slot, row, pl.ds(cg * 16, 16)]
                    iv = ibuf[slot, row, pl.ds(cg * 16, 16)]
                    plsc.addupdate_scatter(acc, [(iv << 7) + iotas[cg]], val)
            return carry2

        lax.fori_loop(0, nrows // 8, _rows, 0)

    for cp in _copies(s * ROWS_PER_TILE, 0):
        cp.start()

    def _chunk(k, carry):
        slot = lax.rem(k, 2)

        @pl.when(k + 1 < NCHUNK)
        def _():
            for cp in _copies(s * ROWS_PER_TILE + (k + 1) * CH,
                              lax.rem(k + 1, 2)):
                cp.start()

        for cp in _copies(s * ROWS_PER_TILE + k * CH, slot):
            cp.wait()
        _compute(slot, CH)
        return carry

    lax.fori_loop(0, NCHUNK, _chunk, 0)

    @pl.when(s < 2)
    def _():
        t0 = TAIL0 + s * TAILCH
        pltpu.sync_copy(src_hbm.at[pl.ds(t0, TAILCH), pl.ds(col0, CW)],
                        sbuf.at[0, pl.ds(0, TAILCH)])
        pltpu.sync_copy(idx_hbm.at[pl.ds(t0, TAILCH), pl.ds(col0, CW)],
                        ibuf.at[0, pl.ds(0, TAILCH)])
        _compute(0, TAILCH)

    pltpu.sync_copy(acc, out_hbm.at[c, s])


def _stage_c(src, index):
    mesh = plsc.VectorSubcoreMesh(core_axis_name="c", subcore_axis_name="s")
    f = pl.kernel(
        _sc_body,
        out_type=jax.ShapeDtypeStruct((NC, NS, G * CW), jnp.float32),
        mesh=mesh,
        compiler_params=pltpu.CompilerParams(needs_layout_passes=False),
        scratch_types=[
            pltpu.VMEM((2, CH, CW), jnp.float32),
            pltpu.VMEM((2, CH, CW), jnp.int32),
            pltpu.VMEM((G * CW,), jnp.float32),
            pltpu.SemaphoreType.DMA((2,)),
        ],
    )
    return f(src, index)


def _stage_d_body(p_ref, o_ref):
    o_ref[...] = jnp.tanh(jnp.sum(p_ref[0], axis=0))


def _stage_d(partial4):
    return pl.pallas_call(
        _stage_d_body,
        grid=(NC,),
        in_specs=[pl.BlockSpec((1, NS, G, CW), lambda c: (c, 0, 0, 0))],
        out_specs=pl.BlockSpec((G, CW), lambda c: (0, c)),
        out_shape=jax.ShapeDtypeStruct((G, D_OUT), jnp.float32),
    )(partial4)


def kernel(x, n_graph, index, Wg, bg, W, b):
    b2 = b.reshape(1, D_OUT)
    bg2 = bg.reshape(1, 1)
    wgp = jnp.pad(Wg, ((0, 127), (0, 0)))
    h, gnum, denom = _stage_a(x, index, W, b2, wgp, bg2)
    src = _stage_b(h, index, gnum, denom)
    partial = _stage_c(src, index)
    return _stage_d(partial.reshape(NC, NS, G, CW))


# speedup vs baseline: 1.5406x; 1.0163x over previous
import jax
import jax.numpy as jnp
from jax import lax
from jax.experimental import pallas as pl
from jax.experimental.pallas import tpu as pltpu
from jax.experimental.pallas import tpu_sc as plsc

N = 100000
D_IN = 256
D_OUT = 256
G = 512
BN = 2000
GRID = N // BN
NC = 2
NS = 16
CH = 120
ROWS_PER_TILE = 6240
NCHUNK = ROWS_PER_TILE // CH
TAIL0 = NS * ROWS_PER_TILE
TAILCH = 80
CW = 128


def _stage_a_body(x_ref, idx_ref, w_ref, b_ref, wg_ref, bg_ref,
                  h_ref, gnum_ref, denom_ref):
    x = x_ref[...]
    xb = x.astype(jnp.bfloat16)
    h_ref[...] = jnp.tanh(
        lax.dot_general(xb, w_ref[...].astype(jnp.bfloat16),
                        (((1,), (1,)), ((), ())),
                        preferred_element_type=jnp.float32) + b_ref[...])
    gfull = lax.dot_general(x, wg_ref[...], (((1,), (1,)), ((), ())),
                            preferred_element_type=jnp.float32)
    gnum = jnp.exp(gfull[:, :1] + bg_ref[0, 0])
    gnum_ref[...] = gnum
    seg = idx_ref[:, :1]
    oh = (lax.broadcasted_iota(jnp.int32, (BN, G), 1) == seg
          ).astype(jnp.float32)
    part = lax.dot_general(oh, gnum, (((0,), (0,)), ((), ())),
                           preferred_element_type=jnp.float32)

    @pl.when(pl.program_id(0) == 0)
    def _():
        denom_ref[...] = part

    @pl.when(pl.program_id(0) != 0)
    def _():
        denom_ref[...] += part


def _stage_a(x, index, w, b2, wg, bg2):
    return pl.pallas_call(
        _stage_a_body,
        grid=(GRID,),
        in_specs=[
            pl.BlockSpec((BN, D_IN), lambda i: (i, 0)),
            pl.BlockSpec((BN, 128), lambda i: (i, 0)),
            pl.BlockSpec((D_OUT, D_IN), lambda i: (0, 0)),
            pl.BlockSpec((1, D_OUT), lambda i: (0, 0)),
            pl.BlockSpec((128, D_IN), lambda i: (0, 0)),
            pl.BlockSpec(memory_space=pltpu.SMEM),
        ],
        out_specs=[
            pl.BlockSpec((BN, D_OUT), lambda i: (i, 0)),
            pl.BlockSpec((BN, 1), lambda i: (i, 0)),
            pl.BlockSpec((G, 1), lambda i: (0, 0)),
        ],
        out_shape=[
            jax.ShapeDtypeStruct((N, D_OUT), jnp.float32),
            jax.ShapeDtypeStruct((N, 1), jnp.float32),
            jax.ShapeDtypeStruct((G, 1), jnp.float32),
        ],
    )(x, index, w, b2, wg, bg2)


def _stage_b_body(h_ref, idx_ref, gnum_ref, denom_ref, src_ref):
    d = denom_ref[...]
    rec = jnp.where(d > 0.0, 1.0 / d, 0.0)
    seg = idx_ref[:, :1]
    oh = (lax.broadcasted_iota(jnp.int32, (BN, G), 1) == seg
          ).astype(jnp.float32)
    gathered = lax.dot_general(oh, rec, (((1,), (0,)), ((), ())),
                               preferred_element_type=jnp.float32)
    src_ref[...] = (gnum_ref[...] * gathered) * h_ref[...]


def _stage_b(h, index, gnum, denom):
    return pl.pallas_call(
        _stage_b_body,
        grid=(GRID,),
        in_specs=[
            pl.BlockSpec((BN, D_OUT), lambda i: (i, 0)),
            pl.BlockSpec((BN, 128), lambda i: (i, 0)),
            pl.BlockSpec((BN, 1), lambda i: (i, 0)),
            pl.BlockSpec((G, 1), lambda i: (0, 0)),
        ],
        out_specs=pl.BlockSpec((BN, D_OUT), lambda i: (i, 0)),
        out_shape=jax.ShapeDtypeStruct((N, D_OUT), jnp.float32),
    )(h, index, gnum, denom)


def _sc_body(src_hbm, idx_hbm, out_hbm, sbuf, ibuf, acc, sem):
    c = lax.axis_index("c")
    s = lax.axis_index("s")
    col0 = c * CW
    iotas = [lax.iota(jnp.int32, 16) + 16 * cg for cg in range(CW // 16)]
    zero16 = jnp.zeros((16,), jnp.float32)

    def _zero(i, carry):
        for cg in range(CW // 16):
            acc[i, pl.ds(cg * 16, 16)] = zero16
        return carry

    lax.fori_loop(0, G, _zero, 0)

    def _copies(r0, slot):
        return (
            pltpu.make_async_copy(
                src_hbm.at[pl.ds(r0, CH), pl.ds(col0, CW)],
                sbuf.at[slot], sem.at[slot]),
            pltpu.make_async_copy(
                idx_hbm.at[pl.ds(r0, CH), pl.ds(col0, CW)],
                ibuf.at[slot], sem.at[slot]),
        )

    def _compute(slot, nrows):
        def _rows(g, carry2):
            base = g * 8
            for u in range(8):
                row = base + u
                for cg in range(CW // 16):
                    val = sbuf[slot, row, pl.ds(cg * 16, 16)]
                    iv = ibuf[slot, row, pl.ds(cg * 16, 16)]
                    plsc.addupdate_scatter(acc, [iv, iotas[cg]], val)
            return carry2

        lax.fori_loop(0, nrows // 8, _rows, 0)

    for cp in _copies(s * ROWS_PER_TILE, 0):
        cp.start()

    def _chunk(k, carry):
        slot = lax.rem(k, 2)

        @pl.when(k + 1 < NCHUNK)
        def _():
            for cp in _copies(s * ROWS_PER_TILE + (k + 1) * CH,
                              lax.rem(k + 1, 2)):
                cp.start()

        for cp in _copies(s * ROWS_PER_TILE + k * CH, slot):
            cp.wait()
        _compute(slot, CH)
        return carry

    lax.fori_loop(0, NCHUNK, _chunk, 0)

    @pl.when(s < 2)
    def _():
        t0 = TAIL0 + s * TAILCH
        pltpu.sync_copy(src_hbm.at[pl.ds(t0, TAILCH), pl.ds(col0, CW)],
                        sbuf.at[0, pl.ds(0, TAILCH)])
        pltpu.sync_copy(idx_hbm.at[pl.ds(t0, TAILCH), pl.ds(col0, CW)],
                        ibuf.at[0, pl.ds(0, TAILCH)])
        _compute(0, TAILCH)

    pltpu.sync_copy(acc, out_hbm.at[c, s])


def _stage_c(src, index):
    mesh = plsc.VectorSubcoreMesh(core_axis_name="c", subcore_axis_name="s")
    f = pl.kernel(
        _sc_body,
        out_type=jax.ShapeDtypeStruct((NC, NS, G, CW), jnp.float32),
        mesh=mesh,
        compiler_params=pltpu.CompilerParams(needs_layout_passes=False),
        scratch_types=[
            pltpu.VMEM((2, CH, CW), jnp.float32),
            pltpu.VMEM((2, CH, CW), jnp.int32),
            pltpu.VMEM((G, CW), jnp.float32),
            pltpu.SemaphoreType.DMA((2,)),
        ],
    )
    return f(src, index)


def _stage_d_body(p_ref, o_ref):
    o_ref[...] = jnp.tanh(jnp.sum(p_ref[0], axis=0))


def _stage_d(partial4):
    return pl.pallas_call(
        _stage_d_body,
        grid=(NC,),
        in_specs=[pl.BlockSpec((1, NS, G, CW), lambda c: (c, 0, 0, 0))],
        out_specs=pl.BlockSpec((G, CW), lambda c: (0, c)),
        out_shape=jax.ShapeDtypeStruct((G, D_OUT), jnp.float32),
    )(partial4)


def kernel(x, n_graph, index, Wg, bg, W, b):
    b2 = b.reshape(1, D_OUT)
    bg2 = bg.reshape(1, 1)
    wgp = jnp.pad(Wg, ((0, 127), (0, 0)))
    h, gnum, denom = _stage_a(x, index, W, b2, wgp, bg2)
    src = _stage_b(h, index, gnum, denom)
    partial = _stage_c(src, index)
    return _stage_d(partial)


# speedup vs baseline: 2.6012x; 1.6884x over previous
import jax
import jax.numpy as jnp
from jax import lax
from jax.experimental import pallas as pl
from jax.experimental.pallas import tpu as pltpu
from jax.experimental.pallas import tpu_sc as plsc

N = 100000
D_IN = 256
D_OUT = 256
G = 512
BN = 2000
GRID = N // BN
NC = 2
NS = 16
CH = 120
ROWS_PER_TILE = 6240
NCHUNK = ROWS_PER_TILE // CH
TAIL0 = NS * ROWS_PER_TILE
TAILCH = 80
CW = 128


def _stage_a_body(x_ref, idx_ref, w_ref, b_ref, wg_ref, bg_ref,
                  h_ref, gnum_ref, denom_ref):
    x = x_ref[...]
    xb = x.astype(jnp.bfloat16)
    h_ref[...] = jnp.tanh(
        lax.dot_general(xb, w_ref[...].astype(jnp.bfloat16),
                        (((1,), (1,)), ((), ())),
                        preferred_element_type=jnp.float32) + b_ref[...])
    gfull = lax.dot_general(x, wg_ref[...], (((1,), (1,)), ((), ())),
                            preferred_element_type=jnp.float32)
    gnum = jnp.exp(gfull[:, :1] + bg_ref[0, 0])
    gnum_ref[...] = gnum
    seg = idx_ref[:, :1]
    oh = (lax.broadcasted_iota(jnp.int32, (BN, G), 1) == seg
          ).astype(jnp.float32)
    part = lax.dot_general(oh, gnum, (((0,), (0,)), ((), ())),
                           preferred_element_type=jnp.float32)

    @pl.when(pl.program_id(0) == 0)
    def _():
        denom_ref[...] = part

    @pl.when(pl.program_id(0) != 0)
    def _():
        denom_ref[...] += part


def _stage_a(x, index, w, b2, wg, bg2):
    return pl.pallas_call(
        _stage_a_body,
        grid=(GRID,),
        in_specs=[
            pl.BlockSpec((BN, D_IN), lambda i: (i, 0)),
            pl.BlockSpec((BN, 128), lambda i: (i, 0)),
            pl.BlockSpec((D_OUT, D_IN), lambda i: (0, 0)),
            pl.BlockSpec((1, D_OUT), lambda i: (0, 0)),
            pl.BlockSpec((128, D_IN), lambda i: (0, 0)),
            pl.BlockSpec(memory_space=pltpu.SMEM),
        ],
        out_specs=[
            pl.BlockSpec((BN, D_OUT), lambda i: (i, 0)),
            pl.BlockSpec((BN, 1), lambda i: (i, 0)),
            pl.BlockSpec((G, 1), lambda i: (0, 0)),
        ],
        out_shape=[
            jax.ShapeDtypeStruct((N, D_OUT), jnp.float32),
            jax.ShapeDtypeStruct((N, 1), jnp.float32),
            jax.ShapeDtypeStruct((G, 1), jnp.float32),
        ],
    )(x, index, w, b2, wg, bg2)


def _stage_b_body(h_ref, idx_ref, gnum_ref, denom_ref, src_ref):
    d = denom_ref[...]
    rec = jnp.where(d > 0.0, 1.0 / d, 0.0)
    seg = idx_ref[:, :1]
    oh = (lax.broadcasted_iota(jnp.int32, (BN, G), 1) == seg
          ).astype(jnp.float32)
    gathered = lax.dot_general(oh, rec, (((1,), (0,)), ((), ())),
                               preferred_element_type=jnp.float32)
    src_ref[...] = (gnum_ref[...] * gathered) * h_ref[...]


def _stage_b(h, index, gnum, denom):
    return pl.pallas_call(
        _stage_b_body,
        grid=(GRID,),
        in_specs=[
            pl.BlockSpec((BN, D_OUT), lambda i: (i, 0)),
            pl.BlockSpec((BN, 128), lambda i: (i, 0)),
            pl.BlockSpec((BN, 1), lambda i: (i, 0)),
            pl.BlockSpec((G, 1), lambda i: (0, 0)),
        ],
        out_specs=pl.BlockSpec((BN, D_OUT), lambda i: (i, 0)),
        out_shape=jax.ShapeDtypeStruct((N, D_OUT), jnp.float32),
    )(h, index, gnum, denom)


def _sc_body(src_hbm, idx_hbm, out_hbm, sbuf, ibuf, acc, sem):
    c = lax.axis_index("c")
    s = lax.axis_index("s")
    col0 = c * CW
    iotas = [lax.iota(jnp.int32, 16) + 16 * cg for cg in range(CW // 16)]
    zero16 = jnp.zeros((16,), jnp.float32)

    def _zero(i, carry):
        for cg in range(CW // 16):
            acc[i, pl.ds(cg * 16, 16)] = zero16
        return carry

    lax.fori_loop(0, G, _zero, 0)

    def _copies(r0, slot):
        return (
            pltpu.make_async_copy(
                src_hbm.at[pl.ds(r0, CH), pl.ds(col0, CW)],
                sbuf.at[slot], sem.at[slot]),
            pltpu.make_async_copy(
                idx_hbm.at[pl.ds(r0, CH), pl.ds(col0, CW)],
                ibuf.at[slot], sem.at[slot]),
        )

    def _compute(slot, nrows):
        @plsc.parallel_loop(0, nrows, 1, unroll=8)
        def _row(row):
            for cg in range(CW // 16):
                val = sbuf[slot, row, pl.ds(cg * 16, 16)]
                iv = ibuf[slot, row, pl.ds(cg * 16, 16)]
                plsc.addupdate_scatter(acc, [iv, iotas[cg]], val)

    for cp in _copies(s * ROWS_PER_TILE, 0):
        cp.start()

    def _chunk(k, carry):
        slot = lax.rem(k, 2)

        @pl.when(k + 1 < NCHUNK)
        def _():
            for cp in _copies(s * ROWS_PER_TILE + (k + 1) * CH,
                              lax.rem(k + 1, 2)):
                cp.start()

        for cp in _copies(s * ROWS_PER_TILE + k * CH, slot):
            cp.wait()
        _compute(slot, CH)
        return carry

    lax.fori_loop(0, NCHUNK, _chunk, 0)

    @pl.when(s < 2)
    def _():
        t0 = TAIL0 + s * TAILCH
        pltpu.sync_copy(src_hbm.at[pl.ds(t0, TAILCH), pl.ds(col0, CW)],
                        sbuf.at[0, pl.ds(0, TAILCH)])
        pltpu.sync_copy(idx_hbm.at[pl.ds(t0, TAILCH), pl.ds(col0, CW)],
                        ibuf.at[0, pl.ds(0, TAILCH)])
        _compute(0, TAILCH)

    pltpu.sync_copy(acc, out_hbm.at[c, s])


def _stage_c(src, index):
    mesh = plsc.VectorSubcoreMesh(core_axis_name="c", subcore_axis_name="s")
    f = pl.kernel(
        _sc_body,
        out_type=jax.ShapeDtypeStruct((NC, NS, G, CW), jnp.float32),
        mesh=mesh,
        compiler_params=pltpu.CompilerParams(needs_layout_passes=False),
        scratch_types=[
            pltpu.VMEM((2, CH, CW), jnp.float32),
            pltpu.VMEM((2, CH, CW), jnp.int32),
            pltpu.VMEM((G, CW), jnp.float32),
            pltpu.SemaphoreType.DMA((2,)),
        ],
    )
    return f(src, index)


def _stage_d_body(p_ref, o_ref):
    o_ref[...] = jnp.tanh(jnp.sum(p_ref[0], axis=0))


def _stage_d(partial4):
    return pl.pallas_call(
        _stage_d_body,
        grid=(NC,),
        in_specs=[pl.BlockSpec((1, NS, G, CW), lambda c: (c, 0, 0, 0))],
        out_specs=pl.BlockSpec((G, CW), lambda c: (0, c)),
        out_shape=jax.ShapeDtypeStruct((G, D_OUT), jnp.float32),
    )(partial4)


def kernel(x, n_graph, index, Wg, bg, W, b):
    b2 = b.reshape(1, D_OUT)
    bg2 = bg.reshape(1, 1)
    wgp = jnp.pad(Wg, ((0, 127), (0, 0)))
    h, gnum, denom = _stage_a(x, index, W, b2, wgp, bg2)
    src = _stage_b(h, index, gnum, denom)
    partial = _stage_c(src, index)
    return _stage_d(partial)


# speedup vs baseline: 2.8636x; 1.1009x over previous
import jax
import jax.numpy as jnp
from jax import lax
from jax.experimental import pallas as pl
from jax.experimental.pallas import tpu as pltpu
from jax.experimental.pallas import tpu_sc as plsc

N = 100000
D_IN = 256
D_OUT = 256
G = 512
BN = 2000
GRID = N // BN
NC = 2
NS = 16
CH = 120
ROWS_PER_TILE = 6240
NCHUNK = ROWS_PER_TILE // CH
TAIL0 = NS * ROWS_PER_TILE
TAILCH = 80
CW = 128


def _stage_a_body(x_ref, idx_ref, w_ref, b_ref, wg_ref, bg_ref,
                  h_ref, gnum_ref, denom_ref):
    x = x_ref[...]
    xb = x.astype(jnp.bfloat16)
    h_ref[...] = jnp.tanh(
        lax.dot_general(xb, w_ref[...].astype(jnp.bfloat16),
                        (((1,), (1,)), ((), ())),
                        preferred_element_type=jnp.float32) + b_ref[...])
    gfull = lax.dot_general(wg_ref[...], x, (((1,), (1,)), ((), ())),
                            preferred_element_type=jnp.float32)
    gnum = jnp.exp(gfull[:1, :] + bg_ref[0, 0])
    gnum_ref[...] = gnum[None]
    seg = idx_ref[:, :1]
    oh = (lax.broadcasted_iota(jnp.int32, (BN, G), 1) == seg
          ).astype(jnp.float32)
    part = lax.dot_general(gnum, oh, (((1,), (0,)), ((), ())),
                           preferred_element_type=jnp.float32)

    @pl.when(pl.program_id(0) == 0)
    def _():
        denom_ref[...] = part

    @pl.when(pl.program_id(0) != 0)
    def _():
        denom_ref[...] += part


def _stage_a(x, index, w, b2, wg, bg2):
    return pl.pallas_call(
        _stage_a_body,
        grid=(GRID,),
        in_specs=[
            pl.BlockSpec((BN, D_IN), lambda i: (i, 0)),
            pl.BlockSpec((BN, 128), lambda i: (i, 0)),
            pl.BlockSpec((D_OUT, D_IN), lambda i: (0, 0)),
            pl.BlockSpec((1, D_OUT), lambda i: (0, 0)),
            pl.BlockSpec((128, D_IN), lambda i: (0, 0)),
            pl.BlockSpec(memory_space=pltpu.SMEM),
        ],
        out_specs=[
            pl.BlockSpec((BN, D_OUT), lambda i: (i, 0)),
            pl.BlockSpec((1, 1, BN), lambda i: (i, 0, 0)),
            pl.BlockSpec((1, G), lambda i: (0, 0)),
        ],
        out_shape=[
            jax.ShapeDtypeStruct((N, D_OUT), jnp.float32),
            jax.ShapeDtypeStruct((GRID, 1, BN), jnp.float32),
            jax.ShapeDtypeStruct((1, G), jnp.float32),
        ],
    )(x, index, w, b2, wg, bg2)


def _stage_b_body(h_ref, idx_ref, gnum_ref, denom_ref, src_ref):
    d = denom_ref[...]
    rec = jnp.where(d > 0.0, 1.0 / d, 0.0)
    seg = idx_ref[:, :1]
    oh = (lax.broadcasted_iota(jnp.int32, (BN, G), 1) == seg
          ).astype(jnp.float32)
    gathered = lax.dot_general(oh, rec, (((1,), (1,)), ((), ())),
                               preferred_element_type=jnp.float32)
    gnum_col = jnp.transpose(gnum_ref[0])
    src_ref[...] = (gnum_col * gathered) * h_ref[...]


def _stage_b(h, index, gnum, denom):
    return pl.pallas_call(
        _stage_b_body,
        grid=(GRID,),
        in_specs=[
            pl.BlockSpec((BN, D_OUT), lambda i: (i, 0)),
            pl.BlockSpec((BN, 128), lambda i: (i, 0)),
            pl.BlockSpec((1, 1, BN), lambda i: (i, 0, 0)),
            pl.BlockSpec((1, G), lambda i: (0, 0)),
        ],
        out_specs=pl.BlockSpec((BN, D_OUT), lambda i: (i, 0)),
        out_shape=jax.ShapeDtypeStruct((N, D_OUT), jnp.float32),
    )(h, index, gnum, denom)


def _sc_body(src_hbm, idx_hbm, out_hbm, sbuf, ibuf, acc, sem):
    c = lax.axis_index("c")
    s = lax.axis_index("s")
    col0 = c * CW
    iotas = [lax.iota(jnp.int32, 16) + 16 * cg for cg in range(CW // 16)]
    zero16 = jnp.zeros((16,), jnp.float32)

    def _zero(i, carry):
        for cg in range(CW // 16):
            acc[i, pl.ds(cg * 16, 16)] = zero16
        return carry

    lax.fori_loop(0, G, _zero, 0)

    def _copies(r0, slot):
        return (
            pltpu.make_async_copy(
                src_hbm.at[pl.ds(r0, CH), pl.ds(col0, CW)],
                sbuf.at[slot], sem.at[slot]),
            pltpu.make_async_copy(
                idx_hbm.at[pl.ds(r0, CH), pl.ds(col0, CW)],
                ibuf.at[slot], sem.at[slot]),
        )

    def _compute(slot, nrows):
        @plsc.parallel_loop(0, nrows, 1, unroll=8)
        def _row(row):
            for cg in range(CW // 16):
                val = sbuf[slot, row, pl.ds(cg * 16, 16)]
                iv = ibuf[slot, row, pl.ds(cg * 16, 16)]
                plsc.addupdate_scatter(acc, [iv, iotas[cg]], val)

    for cp in _copies(s * ROWS_PER_TILE, 0):
        cp.start()

    def _chunk(k, carry):
        slot = lax.rem(k, 2)

        @pl.when(k + 1 < NCHUNK)
        def _():
            for cp in _copies(s * ROWS_PER_TILE + (k + 1) * CH,
                              lax.rem(k + 1, 2)):
                cp.start()

        for cp in _copies(s * ROWS_PER_TILE + k * CH, slot):
            cp.wait()
        _compute(slot, CH)
        return carry

    lax.fori_loop(0, NCHUNK, _chunk, 0)

    @pl.when(s < 2)
    def _():
        t0 = TAIL0 + s * TAILCH
        pltpu.sync_copy(src_hbm.at[pl.ds(t0, TAILCH), pl.ds(col0, CW)],
                        sbuf.at[0, pl.ds(0, TAILCH)])
        pltpu.sync_copy(idx_hbm.at[pl.ds(t0, TAILCH), pl.ds(col0, CW)],
                        ibuf.at[0, pl.ds(0, TAILCH)])
        _compute(0, TAILCH)

    pltpu.sync_copy(acc, out_hbm.at[c, s])


def _stage_c(src, index):
    mesh = plsc.VectorSubcoreMesh(core_axis_name="c", subcore_axis_name="s")
    f = pl.kernel(
        _sc_body,
        out_type=jax.ShapeDtypeStruct((NC, NS, G, CW), jnp.float32),
        mesh=mesh,
        compiler_params=pltpu.CompilerParams(needs_layout_passes=False),
        scratch_types=[
            pltpu.VMEM((2, CH, CW), jnp.float32),
            pltpu.VMEM((2, CH, CW), jnp.int32),
            pltpu.VMEM((G, CW), jnp.float32),
            pltpu.SemaphoreType.DMA((2,)),
        ],
    )
    return f(src, index)


def _stage_d_body(p_ref, o_ref):
    o_ref[...] = jnp.tanh(jnp.sum(p_ref[0], axis=0))


def _stage_d(partial4):
    return pl.pallas_call(
        _stage_d_body,
        grid=(NC,),
        in_specs=[pl.BlockSpec((1, NS, G, CW), lambda c: (c, 0, 0, 0))],
        out_specs=pl.BlockSpec((G, CW), lambda c: (0, c)),
        out_shape=jax.ShapeDtypeStruct((G, D_OUT), jnp.float32),
    )(partial4)


def kernel(x, n_graph, index, Wg, bg, W, b):
    b2 = b.reshape(1, D_OUT)
    bg2 = bg.reshape(1, 1)
    wgp = jnp.pad(Wg, ((0, 127), (0, 0)))
    h, gnum, denom = _stage_a(x, index, W, b2, wgp, bg2)
    src = _stage_b(h, index, gnum, denom)
    partial = _stage_c(src, index)
    return _stage_d(partial)


# speedup vs baseline: 2.8972x; 1.0117x over previous
import jax
import jax.numpy as jnp
from jax import lax
from jax.experimental import pallas as pl
from jax.experimental.pallas import tpu as pltpu
from jax.experimental.pallas import tpu_sc as plsc

N = 100000
D_IN = 256
D_OUT = 256
G = 512
BN = 2000
GRID = N // BN
NC = 2
NS = 16
CH = 120
ROWS_PER_TILE = 6240
NCHUNK = ROWS_PER_TILE // CH
TAIL0 = NS * ROWS_PER_TILE
TAILCH = 80
CW = 128


def _stage_a_body(x_ref, idx_ref, w_ref, b_ref, wg_ref, bg_ref,
                  h_ref, gnum_ref, seg_ref, denom_ref):
    x = x_ref[...]
    xb = x.astype(jnp.bfloat16)
    h_ref[...] = jnp.tanh(
        lax.dot_general(xb, w_ref[...].astype(jnp.bfloat16),
                        (((1,), (1,)), ((), ())),
                        preferred_element_type=jnp.float32) + b_ref[...])
    gfull = lax.dot_general(wg_ref[...], x, (((1,), (1,)), ((), ())),
                            preferred_element_type=jnp.float32)
    gnum = jnp.exp(gfull[:1, :] + bg_ref[0, 0])
    gnum_ref[...] = gnum[None]
    seg = idx_ref[:, :1]
    seg_ref[...] = jnp.transpose(seg)[None]
    oh = (lax.broadcasted_iota(jnp.int32, (BN, G), 1) == seg
          ).astype(jnp.float32)
    part = lax.dot_general(gnum, oh, (((1,), (0,)), ((), ())),
                           preferred_element_type=jnp.float32)

    @pl.when(pl.program_id(0) == 0)
    def _():
        denom_ref[...] = part

    @pl.when(pl.program_id(0) != 0)
    def _():
        denom_ref[...] += part


def _stage_a(x, index, w, b2, wg, bg2):
    return pl.pallas_call(
        _stage_a_body,
        grid=(GRID,),
        in_specs=[
            pl.BlockSpec((BN, D_IN), lambda i: (i, 0)),
            pl.BlockSpec((BN, 128), lambda i: (i, 0)),
            pl.BlockSpec((D_OUT, D_IN), lambda i: (0, 0)),
            pl.BlockSpec((1, D_OUT), lambda i: (0, 0)),
            pl.BlockSpec((128, D_IN), lambda i: (0, 0)),
            pl.BlockSpec(memory_space=pltpu.SMEM),
        ],
        out_specs=[
            pl.BlockSpec((BN, D_OUT), lambda i: (i, 0)),
            pl.BlockSpec((1, 1, BN), lambda i: (i, 0, 0)),
            pl.BlockSpec((1, 1, BN), lambda i: (i, 0, 0)),
            pl.BlockSpec((1, G), lambda i: (0, 0)),
        ],
        out_shape=[
            jax.ShapeDtypeStruct((N, D_OUT), jnp.float32),
            jax.ShapeDtypeStruct((GRID, 1, BN), jnp.float32),
            jax.ShapeDtypeStruct((GRID, 1, BN), jnp.int32),
            jax.ShapeDtypeStruct((1, G), jnp.float32),
        ],
    )(x, index, w, b2, wg, bg2)


def _stage_b_body(h_ref, seg_ref, gnum_ref, denom_ref, src_ref):
    d = denom_ref[...]
    rec = jnp.where(d > 0.0, 1.0 / d, 0.0)
    seg = jnp.transpose(seg_ref[0])
    oh = (lax.broadcasted_iota(jnp.int32, (BN, G), 1) == seg
          ).astype(jnp.float32)
    gathered = lax.dot_general(oh, rec, (((1,), (1,)), ((), ())),
                               preferred_element_type=jnp.float32)
    gnum_col = jnp.transpose(gnum_ref[0])
    src_ref[...] = (gnum_col * gathered) * h_ref[...]


def _stage_b(h, seg, gnum, denom):
    return pl.pallas_call(
        _stage_b_body,
        grid=(GRID,),
        in_specs=[
            pl.BlockSpec((BN, D_OUT), lambda i: (i, 0)),
            pl.BlockSpec((1, 1, BN), lambda i: (i, 0, 0)),
            pl.BlockSpec((1, 1, BN), lambda i: (i, 0, 0)),
            pl.BlockSpec((1, G), lambda i: (0, 0)),
        ],
        out_specs=pl.BlockSpec((BN, D_OUT), lambda i: (i, 0)),
        out_shape=jax.ShapeDtypeStruct((N, D_OUT), jnp.float32),
    )(h, seg, gnum, denom)


def _sc_body(src_hbm, idx_hbm, out_hbm, sbuf, ibuf, acc, sem):
    c = lax.axis_index("c")
    s = lax.axis_index("s")
    col0 = c * CW
    iotas = [lax.iota(jnp.int32, 16) + 16 * cg for cg in range(CW // 16)]
    zero16 = jnp.zeros((16,), jnp.float32)

    def _zero(i, carry):
        for cg in range(CW // 16):
            acc[i, pl.ds(cg * 16, 16)] = zero16
        return carry

    lax.fori_loop(0, G, _zero, 0)

    def _copies(r0, slot):
        return (
            pltpu.make_async_copy(
                src_hbm.at[pl.ds(r0, CH), pl.ds(col0, CW)],
                sbuf.at[slot], sem.at[slot]),
            pltpu.make_async_copy(
                idx_hbm.at[pl.ds(r0, CH), pl.ds(col0, CW)],
                ibuf.at[slot], sem.at[slot]),
        )

    def _compute(slot, nrows):
        @plsc.parallel_loop(0, nrows, 1, unroll=8)
        def _row(row):
            for cg in range(CW // 16):
                val = sbuf[slot, row, pl.ds(cg * 16, 16)]
                iv = ibuf[slot, row, pl.ds(cg * 16, 16)]
                plsc.addupdate_scatter(acc, [iv, iotas[cg]], val)

    for cp in _copies(s * ROWS_PER_TILE, 0):
        cp.start()

    def _chunk(k, carry):
        slot = lax.rem(k, 2)

        @pl.when(k + 1 < NCHUNK)
        def _():
            for cp in _copies(s * ROWS_PER_TILE + (k + 1) * CH,
                              lax.rem(k + 1, 2)):
                cp.start()

        for cp in _copies(s * ROWS_PER_TILE + k * CH, slot):
            cp.wait()
        _compute(slot, CH)
        return carry

    lax.fori_loop(0, NCHUNK, _chunk, 0)

    @pl.when(s < 2)
    def _():
        t0 = TAIL0 + s * TAILCH
        pltpu.sync_copy(src_hbm.at[pl.ds(t0, TAILCH), pl.ds(col0, CW)],
                        sbuf.at[0, pl.ds(0, TAILCH)])
        pltpu.sync_copy(idx_hbm.at[pl.ds(t0, TAILCH), pl.ds(col0, CW)],
                        ibuf.at[0, pl.ds(0, TAILCH)])
        _compute(0, TAILCH)

    pltpu.sync_copy(acc, out_hbm.at[c, s])


def _stage_c(src, index):
    mesh = plsc.VectorSubcoreMesh(core_axis_name="c", subcore_axis_name="s")
    f = pl.kernel(
        _sc_body,
        out_type=jax.ShapeDtypeStruct((NC, NS, G, CW), jnp.float32),
        mesh=mesh,
        compiler_params=pltpu.CompilerParams(needs_layout_passes=False),
        scratch_types=[
            pltpu.VMEM((2, CH, CW), jnp.float32),
            pltpu.VMEM((2, CH, CW), jnp.int32),
            pltpu.VMEM((G, CW), jnp.float32),
            pltpu.SemaphoreType.DMA((2,)),
        ],
    )
    return f(src, index)


def _stage_d_body(p_ref, o_ref):
    o_ref[...] = jnp.tanh(jnp.sum(p_ref[0], axis=0))


def _stage_d(partial4):
    return pl.pallas_call(
        _stage_d_body,
        grid=(NC,),
        in_specs=[pl.BlockSpec((1, NS, G, CW), lambda c: (c, 0, 0, 0))],
        out_specs=pl.BlockSpec((G, CW), lambda c: (0, c)),
        out_shape=jax.ShapeDtypeStruct((G, D_OUT), jnp.float32),
    )(partial4)


def kernel(x, n_graph, index, Wg, bg, W, b):
    b2 = b.reshape(1, D_OUT)
    bg2 = bg.reshape(1, 1)
    wgp = jnp.pad(Wg, ((0, 127), (0, 0)))
    h, gnum, seg, denom = _stage_a(x, index, W, b2, wgp, bg2)
    src = _stage_b(h, seg, gnum, denom)
    partial = _stage_c(src, index)
    return _stage_d(partial)
